# baseline (speedup 1.0000x reference)
import functools

import jax
import jax.numpy as jnp
from jax import lax
from jax.experimental import pallas as pl
from jax.experimental.pallas import tpu as pltpu
from jax.experimental.pallas import tpu_sc as plsc

N = 10000
E = 320000
F_IN = 128
H = 512
G = 64

NC = 2
NS = 16
CW = 128
GC = 8
NCH = 2560
EP = NCH * CW
NG2 = NCH // NS // GC
NG1 = NCH // (NC * NS) // GC
RQ = 624
TAIL = N - NS * RQ
BN = 1000

_mesh = plsc.VectorSubcoreMesh(core_axis_name="c", subcore_axis_name="s")


def _scale_chunk(buf, wg, j):
    @pl.loop(0, CW // 16)
    def _(g):
        w16 = wg[j, pl.ds(g * 16, 16)]
        for l in range(16):
            wv = w16[l]
            for jj in range(8):
                sl = (g * 16 + l, pl.ds(16 * jj, 16))
                buf[sl] = buf[sl] * wv


def _run_pass(h_hbm, acc, src_hbm, dst_hbm, w_hbm, gbase, ng, off,
              srcg, dstg, wg, rows, semi, semg, semsc):
    def fire_idx(g):
        sl = pl.ds((gbase + g) * GC, GC)
        pltpu.async_copy(src_hbm.at[sl], srcg, semi)
        pltpu.async_copy(dst_hbm.at[sl], dstg, semi)
        pltpu.async_copy(w_hbm.at[sl], wg, semi)

    def wait_idx(g):
        sl = pl.ds((gbase + g) * GC, GC)
        pltpu.make_async_copy(src_hbm.at[sl], srcg, semi).wait()
        pltpu.make_async_copy(dst_hbm.at[sl], dstg, semi).wait()
        pltpu.make_async_copy(w_hbm.at[sl], wg, semi).wait()

    def add_off():
        if off is not None:
            for rj in range(GC):
                for i in range(CW // 16):
                    sl = (rj, pl.ds(16 * i, 16))
                    srcg[sl] = srcg[sl] + off

    def fire_gather(rj, b):
        pltpu.async_copy(h_hbm.at[srcg.at[rj]], rows[b], semg[b])

    def wait_gather(rj, b):
        pltpu.make_async_copy(h_hbm.at[srcg.at[rj]], rows[b],
                              semg[b]).wait()

    def fire_scatter(rj, b):
        pltpu.async_copy(rows[b], acc.at[dstg.at[rj]], semsc[b], add=True)

    def wait_scatter(rj, b):
        pltpu.make_async_copy(rows[b], acc.at[dstg.at[rj]],
                              semsc[b]).wait()

    def load_group(g):
        fire_idx(g)
        wait_idx(g)
        add_off()
        fire_gather(0, 0)

    load_group(0)

    @pl.loop(0, ng)
    def _(g):
        @pl.loop(0, GC)
        def _(j):
            for par in range(2):
                @pl.when(j % 2 == par)
                def _():
                    b, nb = par, 1 - par
                    wait_gather(j, b)

                    @pl.when(j >= 1)
                    def _():
                        wait_scatter(j - 1, nb)

                    @pl.when(j < GC - 1)
                    def _():
                        fire_gather(j + 1, nb)

                    _scale_chunk(rows[b], wg, j)
                    fire_scatter(j, b)

        @pl.when(g + 1 < ng)
        def _():
            wait_scatter(GC - 1, 1)
            load_group(g + 1)

    wait_scatter(GC - 1, 1)


def _zero_acc(z_hbm, acc, s):
    r0 = s * RQ
    pltpu.sync_copy(z_hbm.at[pl.ds(r0, RQ)], acc.at[pl.ds(r0, RQ)])

    @pl.when(s == NS - 1)
    def _():
        pltpu.sync_copy(z_hbm.at[pl.ds(NS * RQ, TAIL)],
                        acc.at[pl.ds(NS * RQ, TAIL)])


def _sc_scratch():
    return ([pltpu.VMEM_SHARED((N, 128), jnp.float32)]
            + [pltpu.VMEM((GC, CW), jnp.int32) for _ in range(2)]
            + [pltpu.VMEM((GC, CW), jnp.float32)]
            + [pltpu.VMEM((CW, 128), jnp.float32) for _ in range(2)]
            + [pltpu.SemaphoreType.DMA for _ in range(5)])


def _split_bufs(bufs):
    srcg, dstg, wg = bufs[0], bufs[1], bufs[2]
    rows = list(bufs[3:5])
    semi = bufs[5]
    semg = list(bufs[6:8])
    semsc = list(bufs[8:10])
    return srcg, dstg, wg, rows, semi, semg, semsc


@functools.partial(
    pl.kernel,
    out_type=jax.ShapeDtypeStruct((NC, N, F_IN), jnp.float32),
    mesh=_mesh,
    scratch_types=_sc_scratch(),
)
def _sc_agg1(x_hbm, src_hbm, dst_hbm, w_hbm, z_hbm, out_hbm, acc, *bufs):
    srcg, dstg, wg, rows, semi, semg, semsc = _split_bufs(bufs)
    c = lax.axis_index("c")
    s = lax.axis_index("s")
    gbase = c * (NCH // NC // GC) + s * NG1
    _zero_acc(z_hbm, acc, s)
    plsc.subcore_barrier()

    _run_pass(x_hbm, acc, src_hbm, dst_hbm, w_hbm, gbase, NG1, None,
              srcg, dstg, wg, rows, semi, semg, semsc)

    plsc.subcore_barrier()
    r0 = s * RQ
    pltpu.sync_copy(acc.at[pl.ds(r0, RQ)], out_hbm.at[c, pl.ds(r0, RQ)])

    @pl.when(s == NS - 1)
    def _():
        pltpu.sync_copy(acc.at[pl.ds(NS * RQ, TAIL)],
                        out_hbm.at[c, pl.ds(NS * RQ, TAIL)])


@functools.partial(
    pl.kernel,
    out_type=jax.ShapeDtypeStruct((N, H), jnp.float32),
    mesh=_mesh,
    scratch_types=_sc_scratch(),
)
def _sc_agg2(h_hbm, src_hbm, dst_hbm, w_hbm, z_hbm, out_hbm, acc, *bufs):
    srcg, dstg, wg, rows, semi, semg, semsc = _split_bufs(bufs)
    c = lax.axis_index("c")
    s = lax.axis_index("s")
    gbase = s * NG2

    for p in range(2):
        cc = 2 * c + p
        _zero_acc(z_hbm, acc, s)
        plsc.subcore_barrier()

        _run_pass(h_hbm, acc, src_hbm, dst_hbm, w_hbm, gbase, NG2, cc * N,
                  srcg, dstg, wg, rows, semi, semg, semsc)

        plsc.subcore_barrier()
        r0 = s * RQ
        pltpu.sync_copy(acc.at[pl.ds(r0, RQ)],
                        out_hbm.at[pl.ds(r0, RQ), pl.ds(cc * 128, 128)])

        @pl.when(s == NS - 1)
        def _():
            pltpu.sync_copy(acc.at[pl.ds(NS * RQ, TAIL)],
                            out_hbm.at[pl.ds(NS * RQ, TAIL),
                                       pl.ds(cc * 128, 128)])
        plsc.subcore_barrier()


def _tc1(p, x, W_rel1, b1, W_root1):
    def body(p_ref, x_ref, wr, bb, wo, o_ref):
        agg = p_ref[0] + p_ref[1]
        h = jnp.dot(agg, wr[...], preferred_element_type=jnp.float32)
        h = h + bb[...] + jnp.dot(x_ref[...], wo[...],
                                  preferred_element_type=jnp.float32)
        h = jnp.maximum(h, 0.0)
        for cc in range(4):
            o_ref[cc] = h[:, 128 * cc:128 * (cc + 1)]

    return pl.pallas_call(
        body,
        grid=(N // BN,),
        in_specs=[
            pl.BlockSpec((NC, BN, F_IN), lambda i: (0, i, 0)),
            pl.BlockSpec((BN, F_IN), lambda i: (i, 0)),
            pl.BlockSpec((F_IN, H), lambda i: (0, 0)),
            pl.BlockSpec((1, H), lambda i: (0, 0)),
            pl.BlockSpec((F_IN, H), lambda i: (0, 0)),
        ],
        out_specs=pl.BlockSpec((4, BN, 128), lambda i: (0, i, 0)),
        out_shape=jax.ShapeDtypeStruct((4, N, 128), jnp.float32),
    )(p, x, W_rel1, b1, W_root1)


def _tc2(agg2, h1c, batchf, W_rel2, b2, W_root2):
    def body(a_ref, h1_ref, bt_ref, wr, bb, wo, pool_ref):
        i = pl.program_id(0)
        h2 = jnp.dot(a_ref[...], wr[...],
                     preferred_element_type=jnp.float32) + bb[...]
        for cc in range(4):
            h2 = h2 + jnp.dot(h1_ref[cc], wo[pl.ds(128 * cc, 128), :],
                              preferred_element_type=jnp.float32)
        h2 = jnp.maximum(h2, 0.0)
        bt = bt_ref[...]
        gid = lax.broadcasted_iota(jnp.int32, (1, G), 1).astype(jnp.float32)
        onehot = (bt == gid).astype(jnp.float32)
        pool = lax.dot_general(onehot, h2, (((0,), (0,)), ((), ())),
                               preferred_element_type=jnp.float32)

        @pl.when(i == 0)
        def _():
            pool_ref[...] = pool

        @pl.when(i > 0)
        def _():
            pool_ref[...] = pool_ref[...] + pool

    return pl.pallas_call(
        body,
        grid=(N // BN,),
        in_specs=[
            pl.BlockSpec((BN, H), lambda i: (i, 0)),
            pl.BlockSpec((4, BN, 128), lambda i: (0, i, 0)),
            pl.BlockSpec((BN, 1), lambda i: (i, 0)),
            pl.BlockSpec((H, H), lambda i: (0, 0)),
            pl.BlockSpec((1, H), lambda i: (0, 0)),
            pl.BlockSpec((H, H), lambda i: (0, 0)),
        ],
        out_specs=pl.BlockSpec((G, H), lambda i: (0, 0)),
        out_shape=jax.ShapeDtypeStruct((G, H), jnp.float32),
    )(agg2, h1c, batchf, W_rel2, b2, W_root2)


def _tc3(pool, batchf, W_l1, b_l1, W_l2, b_l2, W_out, b_out):
    def body(pool_ref, bt_ref, w1, b1, w2, b2, w3, b3, o_ref):
        bt = bt_ref[...]
        gid = lax.broadcasted_iota(jnp.int32, (1, G), 1).astype(jnp.float32)
        onehot = (bt == gid).astype(jnp.float32)
        ones = jnp.ones((N, 1), jnp.float32)
        cnt = lax.dot_general(onehot, ones, (((0,), (0,)), ((), ())),
                              preferred_element_type=jnp.float32)
        mean = pool_ref[...] / jnp.maximum(cnt, 1.0)
        z = jnp.maximum(jnp.dot(mean, w1[...],
                                preferred_element_type=jnp.float32)
                        + b1[...], 0.0)
        z = jnp.dot(z, w2[...], preferred_element_type=jnp.float32) + b2[...]
        z = jnp.maximum(z, 0.0)
        o_ref[...] = jnp.dot(z, w3[...],
                             preferred_element_type=jnp.float32) + b3[...]

    return pl.pallas_call(
        body,
        out_shape=jax.ShapeDtypeStruct((G, 1), jnp.float32),
    )(pool, batchf, W_l1, b_l1, W_l2, b_l2, W_out, b_out)


def kernel(x, edge_index, edge_attr, batch,
           W_rel1, b_rel1, W_root1,
           W_rel2, b_rel2, W_root2,
           W_l1, b_l1, W_l2, b_l2, W_out, b_out):
    pad = EP - E
    src2 = jnp.concatenate(
        [edge_index[0], jnp.zeros((pad,), jnp.int32)]).reshape(NCH, CW)
    dst2 = jnp.concatenate(
        [edge_index[1], jnp.zeros((pad,), jnp.int32)]).reshape(NCH, CW)
    w2 = jnp.concatenate(
        [edge_attr, jnp.zeros((pad,), jnp.float32)]).reshape(NCH, CW)
    zeros = jnp.zeros((N, 128), jnp.float32)
    batchf = batch.astype(jnp.float32).reshape(N, 1)

    p = _sc_agg1(x, src2, dst2, w2, zeros)
    h1c = _tc1(p, x, W_rel1, b_rel1.reshape(1, H), W_root1)
    agg2 = _sc_agg2(h1c.reshape(4 * N, 128), src2, dst2, w2, zeros)
    pool = _tc2(agg2, h1c, batchf, W_rel2, b_rel2.reshape(1, H), W_root2)
    out = _tc3(pool, batchf, W_l1, b_l1.reshape(1, G), W_l2,
               b_l2.reshape(1, 16), W_out, b_out.reshape(1, 1))
    return out
